# baseline (speedup 1.0000x reference)
import jax
import jax.numpy as jnp
from jax import lax
from jax.experimental import pallas as pl
from jax.experimental.pallas import tpu as pltpu
from jax.experimental.pallas import tpu_sc as plsc

P_DEG = 2
DIM = 3
NC = 2
NS = 16
O_PTS = 2048
CHUNK = 128
NW = O_PTS // CHUNK


def _sc_build_at(nut_hbm, idx_hbm, at_hbm, idx_v, nu_v, at_v):
    wid = lax.axis_index("s") * NC + lax.axis_index("c")

    @pl.when(wid < NW)
    def _():
        base = wid * CHUNK
        pltpu.sync_copy(idx_hbm.at[pl.ds(base, CHUNK)], idx_v)
        pltpu.sync_copy(nut_hbm.at[:, pl.ds(base, CHUNK)], nu_v)

        zeros = jnp.zeros((16,), jnp.float32)

        def _zero_row(r, carry):
            for c in range(CHUNK // 16):
                at_v[r, pl.ds(c * 16, 16)] = zeros
            return carry

        lax.fori_loop(0, 128, _zero_row, 0)

        for c in range(CHUNK // 16):
            i0 = idx_v[pl.ds(c * 16, 16)] - P_DEG
            nuj = [nu_v[j, pl.ds(c * 16, 16)] for j in range(P_DEG + 1)]
            r_lo = i0[0]
            width = i0[15] - i0[0] + P_DEG

            def _value_at(r, i0=i0, nuj=nuj):
                v = jnp.zeros((16,), jnp.float32)
                for j in range(P_DEG + 1):
                    v = jnp.where(i0 + j == r, nuj[j], v)
                return v

            @pl.when(width < 16)
            def _fast(c=c, r_lo=r_lo, width=width, _value_at=_value_at):
                def _band_row(k, carry):
                    @pl.when(k <= width)
                    def _():
                        r = r_lo + k
                        at_v[r, pl.ds(c * 16, 16)] = _value_at(r)
                    return carry

                lax.fori_loop(0, 16, _band_row, 0)

            @pl.when(width >= 16)
            def _general(c=c, _value_at=_value_at):
                def _row(r, carry):
                    at_v[r, pl.ds(c * 16, 16)] = _value_at(r)
                    return carry

                lax.fori_loop(0, 128, _row, 0)

        pltpu.sync_copy(at_v, at_hbm.at[:, pl.ds(base, CHUNK)])


def _blend_body(at_ref, xtt_ref, out_ref):
    at = at_ref[...]
    c = [jnp.dot(xtt_ref[d], at, preferred_element_type=jnp.float32)
         for d in range(DIM + 1)]
    inv = 1.0 / c[DIM]
    out_ref[...] = jnp.stack([c[d] * inv for d in range(DIM)], axis=0)


def kernel(input, Nu, uspan):
    B, M, D1 = input.shape
    O = Nu.shape[0]
    idx1d = uspan.astype(jnp.int32)
    nut = jnp.transpose(Nu, (1, 0))
    xtt = jnp.transpose(input, (2, 0, 1))

    mesh = plsc.VectorSubcoreMesh(core_axis_name="c", subcore_axis_name="s",
                                  num_cores=NC, num_subcores=NS)
    at = pl.kernel(
        _sc_build_at,
        out_type=jax.ShapeDtypeStruct((M, O), jnp.float32),
        mesh=mesh,
        scratch_types=[
            pltpu.VMEM((CHUNK,), jnp.int32),
            pltpu.VMEM((P_DEG + 1, CHUNK), jnp.float32),
            pltpu.VMEM((128, CHUNK), jnp.float32),
        ],
    )(nut, idx1d)

    BT = 128
    NSPLIT = 2
    BH = B // NSPLIT
    halves = []
    for s in range(NSPLIT):
        xs = lax.slice_in_dim(xtt, s * BH, (s + 1) * BH, axis=1)
        out3s = pl.pallas_call(
            _blend_body,
            grid=(BH // BT,),
            in_specs=[
                pl.BlockSpec((M, O), lambda i: (0, 0)),
                pl.BlockSpec((D1, BT, M), lambda i: (0, i, 0)),
            ],
            out_specs=pl.BlockSpec((DIM, BT, O), lambda i: (0, i, 0)),
            out_shape=jax.ShapeDtypeStruct((DIM, BH, O), jnp.float32),
        )(at, xs)
        halves.append(jnp.transpose(out3s, (1, 2, 0)))
    return jnp.concatenate(halves, axis=0)


# speedup vs baseline: 1.0508x; 1.0508x over previous
import jax
import jax.numpy as jnp
from jax import lax
from jax.experimental import pallas as pl
from jax.experimental.pallas import tpu as pltpu
from jax.experimental.pallas import tpu_sc as plsc

P_DEG = 2
DIM = 3
NC = 2
NS = 16
O_PTS = 2048
CHUNK = 128
NW = O_PTS // CHUNK


def _sc_build_at(nut_hbm, idx_hbm, at_hbm, idx_v, nu_v, at_v, sem_i, sem_n):
    wid = lax.axis_index("s")

    @pl.when(wid < NW)
    def _():
        base = wid * CHUNK
        cp_i = pltpu.make_async_copy(idx_hbm.at[pl.ds(base, CHUNK)], idx_v, sem_i)
        cp_n = pltpu.make_async_copy(nut_hbm.at[:, pl.ds(base, CHUNK)], nu_v, sem_n)
        cp_i.start()
        cp_n.start()

        zeros = jnp.zeros((16,), jnp.float32)

        def _zero_row(r):
            for c in range(CHUNK // 16):
                at_v[r, pl.ds(c * 16, 16)] = zeros

        plsc.parallel_loop(0, 128, 1, unroll=4)(_zero_row)

        cp_i.wait()
        cp_n.wait()

        for c in range(CHUNK // 16):
            i0 = idx_v[pl.ds(c * 16, 16)] - P_DEG
            nuj = [nu_v[j, pl.ds(c * 16, 16)] for j in range(P_DEG + 1)]
            r_lo = i0[0]
            width = i0[15] - i0[0] + P_DEG

            def _value_at(r, i0=i0, nuj=nuj):
                v = jnp.zeros((16,), jnp.float32)
                for j in range(P_DEG + 1):
                    v = jnp.where(i0 + j == r, nuj[j], v)
                return v

            @pl.when(width < 4)
            def _fastest(c=c, r_lo=r_lo, width=width, _value_at=_value_at):
                for k in range(4):
                    @pl.when(k <= width)
                    def _(k=k):
                        r = r_lo + k
                        at_v[r, pl.ds(c * 16, 16)] = _value_at(r)

            @pl.when(width >= 4)
            def _general(c=c, _value_at=_value_at):
                def _row(r, carry):
                    at_v[r, pl.ds(c * 16, 16)] = _value_at(r)
                    return carry

                lax.fori_loop(0, 128, _row, 0)

        pltpu.sync_copy(at_v, at_hbm.at[:, pl.ds(base, CHUNK)])


def _blend_body(at_ref, xtt_ref, out_ref):
    at = at_ref[...]
    c = [jnp.dot(xtt_ref[d], at, preferred_element_type=jnp.float32)
         for d in range(DIM + 1)]
    inv = 1.0 / c[DIM]
    out_ref[...] = jnp.stack([c[d] * inv for d in range(DIM)], axis=0)


def kernel(input, Nu, uspan):
    B, M, D1 = input.shape
    O = Nu.shape[0]
    idx1d = uspan.astype(jnp.int32)
    nut = jnp.transpose(Nu, (1, 0))
    xtt = jnp.transpose(input, (2, 0, 1))

    mesh = plsc.VectorSubcoreMesh(core_axis_name="c", subcore_axis_name="s",
                                  num_cores=1, num_subcores=NS)
    at = pl.kernel(
        _sc_build_at,
        out_type=jax.ShapeDtypeStruct((M, O), jnp.float32),
        mesh=mesh,
        scratch_types=[
            pltpu.VMEM((CHUNK,), jnp.int32),
            pltpu.VMEM((P_DEG + 1, CHUNK), jnp.float32),
            pltpu.VMEM((128, CHUNK), jnp.float32),
            pltpu.SemaphoreType.DMA,
            pltpu.SemaphoreType.DMA,
        ],
    )(nut, idx1d)

    BT = 128
    NSPLIT = 2
    BH = B // NSPLIT
    halves = []
    for s in range(NSPLIT):
        xs = lax.slice_in_dim(xtt, s * BH, (s + 1) * BH, axis=1)
        out3s = pl.pallas_call(
            _blend_body,
            grid=(BH // BT,),
            in_specs=[
                pl.BlockSpec((M, O), lambda i: (0, 0)),
                pl.BlockSpec((D1, BT, M), lambda i: (0, i, 0)),
            ],
            out_specs=pl.BlockSpec((DIM, BT, O), lambda i: (0, i, 0)),
            out_shape=jax.ShapeDtypeStruct((DIM, BH, O), jnp.float32),
        )(at, xs)
        halves.append(jnp.transpose(out3s, (1, 2, 0)))
    return jnp.concatenate(halves, axis=0)


# speedup vs baseline: 1.6913x; 1.6095x over previous
import jax
import jax.numpy as jnp
from jax import lax
from jax.experimental import pallas as pl
from jax.experimental.pallas import tpu as pltpu
from jax.experimental.pallas import tpu_sc as plsc

P_DEG = 2
DIM = 3
NC = 2
NS = 16
O_PTS = 2048
CHUNK = 128
NW = O_PTS // CHUNK


def _sc_build_at(nut_hbm, idx_hbm, at_hbm, idx_v, nu_v, at_v, sem_i, sem_n):
    wid = lax.axis_index("s")

    @pl.when(wid < NW)
    def _():
        base = wid * CHUNK
        cp_i = pltpu.make_async_copy(idx_hbm.at[pl.ds(base, CHUNK)], idx_v, sem_i)
        cp_n = pltpu.make_async_copy(nut_hbm.at[:, pl.ds(base, CHUNK)], nu_v, sem_n)
        cp_i.start()
        cp_n.start()

        zeros = jnp.zeros((16,), jnp.float32)

        def _zero_row(r):
            for c in range(CHUNK // 16):
                at_v[r, pl.ds(c * 16, 16)] = zeros

        plsc.parallel_loop(0, 128, 1, unroll=4)(_zero_row)

        cp_i.wait()
        cp_n.wait()

        for c in range(CHUNK // 16):
            i0 = idx_v[pl.ds(c * 16, 16)] - P_DEG
            nuj = [nu_v[j, pl.ds(c * 16, 16)] for j in range(P_DEG + 1)]
            r_lo = i0[0]
            width = i0[15] - i0[0] + P_DEG

            def _value_at(r, i0=i0, nuj=nuj):
                v = jnp.zeros((16,), jnp.float32)
                for j in range(P_DEG + 1):
                    v = jnp.where(i0 + j == r, nuj[j], v)
                return v

            @pl.when(width < 4)
            def _fastest(c=c, r_lo=r_lo, width=width, _value_at=_value_at):
                for k in range(4):
                    @pl.when(k <= width)
                    def _(k=k):
                        r = r_lo + k
                        at_v[r, pl.ds(c * 16, 16)] = _value_at(r)

            @pl.when(width >= 4)
            def _general(c=c, _value_at=_value_at):
                def _row(r, carry):
                    at_v[r, pl.ds(c * 16, 16)] = _value_at(r)
                    return carry

                lax.fori_loop(0, 128, _row, 0)

        pltpu.sync_copy(at_v, at_hbm.at[:, pl.ds(base, CHUNK)])


def _blend_body(at_ref, xtt_ref, out_ref):
    at = at_ref[...]
    c = [jnp.dot(xtt_ref[d], at, preferred_element_type=jnp.float32)
         for d in range(DIM + 1)]
    inv = 1.0 / c[DIM]
    out_ref[...] = jnp.stack([c[d] * inv for d in range(DIM)], axis=0)


def kernel(input, Nu, uspan):
    B, M, D1 = input.shape
    O = Nu.shape[0]
    idx1d = uspan.astype(jnp.int32)
    nut = jnp.transpose(Nu, (1, 0))
    xtt = jnp.transpose(input, (2, 0, 1))

    mesh = plsc.VectorSubcoreMesh(core_axis_name="c", subcore_axis_name="s",
                                  num_cores=1, num_subcores=NS)
    at = pl.kernel(
        _sc_build_at,
        out_type=jax.ShapeDtypeStruct((M, O), jnp.float32),
        mesh=mesh,
        scratch_types=[
            pltpu.VMEM((CHUNK,), jnp.int32),
            pltpu.VMEM((P_DEG + 1, CHUNK), jnp.float32),
            pltpu.VMEM((128, CHUNK), jnp.float32),
            pltpu.SemaphoreType.DMA,
            pltpu.SemaphoreType.DMA,
        ],
    )(nut, idx1d)

    BT = 128
    out3 = pl.pallas_call(
        _blend_body,
        grid=(B // BT,),
        in_specs=[
            pl.BlockSpec((M, O), lambda i: (0, 0)),
            pl.BlockSpec((D1, BT, M), lambda i: (0, i, 0)),
        ],
        out_specs=pl.BlockSpec((DIM, BT, O), lambda i: (0, i, 0)),
        out_shape=jax.ShapeDtypeStruct((DIM, B, O), jnp.float32),
    )(at, xtt)

    return jnp.transpose(out3, (1, 2, 0))


# speedup vs baseline: 1.7550x; 1.0377x over previous
import jax
import jax.numpy as jnp
from jax import lax
from jax.experimental import pallas as pl
from jax.experimental.pallas import tpu as pltpu
from jax.experimental.pallas import tpu_sc as plsc

P_DEG = 2
DIM = 3
NC = 2
NS = 16
O_PTS = 2048
CHUNK = 128
NW = O_PTS // CHUNK


def _sc_build_at(nut_hbm, idx_hbm, at_hbm, idx_v, nu_v, at_v, sem_i, sem_n):
    wid = lax.axis_index("s")

    @pl.when(wid < NW)
    def _():
        base = wid * CHUNK
        cp_i = pltpu.make_async_copy(idx_hbm.at[pl.ds(base, CHUNK)], idx_v, sem_i)
        cp_n = pltpu.make_async_copy(nut_hbm.at[:, pl.ds(base, CHUNK)], nu_v, sem_n)
        cp_i.start()
        cp_n.start()

        zeros = jnp.zeros((16,), jnp.float32)

        def _zero_row(r):
            for c in range(CHUNK // 16):
                at_v[r, pl.ds(c * 16, 16)] = zeros

        plsc.parallel_loop(0, 128, 1, unroll=4)(_zero_row)

        cp_i.wait()
        cp_n.wait()

        for c in range(CHUNK // 16):
            i0 = idx_v[pl.ds(c * 16, 16)] - P_DEG
            nuj = [nu_v[j, pl.ds(c * 16, 16)] for j in range(P_DEG + 1)]
            r_lo = i0[0]
            width = i0[15] - i0[0] + P_DEG

            def _value_at(r, i0=i0, nuj=nuj):
                v = jnp.zeros((16,), jnp.float32)
                for j in range(P_DEG + 1):
                    v = jnp.where(i0 + j == r, nuj[j], v)
                return v

            @pl.when(width < 4)
            def _fastest(c=c, r_lo=r_lo, width=width, _value_at=_value_at):
                for k in range(4):
                    @pl.when(k <= width)
                    def _(k=k):
                        r = r_lo + k
                        at_v[r, pl.ds(c * 16, 16)] = _value_at(r)

            @pl.when(width >= 4)
            def _general(c=c, _value_at=_value_at):
                def _row(r, carry):
                    at_v[r, pl.ds(c * 16, 16)] = _value_at(r)
                    return carry

                lax.fori_loop(0, 128, _row, 0)

        pltpu.sync_copy(at_v, at_hbm.at[:, pl.ds(base, CHUNK)])


def _blend_body(at_ref, xtt_ref, out_ref):
    at = at_ref[...]
    c = [jnp.dot(xtt_ref[d], at, preferred_element_type=jnp.float32)
         for d in range(DIM + 1)]
    inv = 1.0 / c[DIM]
    out_ref[...] = jnp.stack([c[d] * inv for d in range(DIM)], axis=0)


def kernel(input, Nu, uspan):
    B, M, D1 = input.shape
    O = Nu.shape[0]
    idx1d = uspan.astype(jnp.int32)
    nut = jnp.transpose(Nu, (1, 0))
    xtt = jnp.transpose(input, (2, 0, 1))

    mesh = plsc.VectorSubcoreMesh(core_axis_name="c", subcore_axis_name="s",
                                  num_cores=1, num_subcores=NS)
    at = pl.kernel(
        _sc_build_at,
        out_type=jax.ShapeDtypeStruct((M, O), jnp.float32),
        mesh=mesh,
        scratch_types=[
            pltpu.VMEM((CHUNK,), jnp.int32),
            pltpu.VMEM((P_DEG + 1, CHUNK), jnp.float32),
            pltpu.VMEM((128, CHUNK), jnp.float32),
            pltpu.SemaphoreType.DMA,
            pltpu.SemaphoreType.DMA,
        ],
    )(nut, idx1d)

    BT = 256
    out3 = pl.pallas_call(
        _blend_body,
        grid=(B // BT,),
        in_specs=[
            pl.BlockSpec((M, O), lambda i: (0, 0)),
            pl.BlockSpec((D1, BT, M), lambda i: (0, i, 0)),
        ],
        out_specs=pl.BlockSpec((DIM, BT, O), lambda i: (0, i, 0)),
        out_shape=jax.ShapeDtypeStruct((DIM, B, O), jnp.float32),
    )(at, xtt)

    return jnp.transpose(out3, (1, 2, 0))
